# baseline (speedup 1.0000x reference)
import jax
import jax.numpy as jnp
import numpy as np
from jax import lax
from jax.experimental import pallas as pl
from jax.experimental.pallas import tpu as pltpu
from jax.experimental.pallas import tpu_sc as plsc

B, T, C = 256, 4096, 64
MAX_SHIFT = 0.1
LIKELIHOOD = 0.2

NC, NS = 2, 16
NW = NC * NS
R_PER = B // NW
NSLAB = C // 8
UNITS = R_PER * NSLAB
NT = T // 128
INV_T = 1.0 / float(T)
SLOT_PAD = 32


def _slot_val(ref, slot):
    return ref[pl.ds(slot, 16)][0]


_GDN = lax.GatherDimensionNumbers(
    offset_dims=(), collapsed_slice_dims=(0,), start_index_map=(0,))


def _shuffle(v, idx):
    return lax.gather(v, idx[:, None], _GDN, (1,),
                      mode=lax.GatherScatterMode.PROMISE_IN_BOUNDS)


def _hsum(v):
    iota = lax.broadcasted_iota(jnp.int32, (16,), 0)
    s = v + lax.rev(v, (0,))
    for m in (4, 2, 1):
        s = s + _shuffle(s, iota ^ m)
    return s[0]


def _sc_body(x_h, rows_h, s_h, out_h, rows_v, sv, buf_a, buf_b,
             rs_a, rs_b, ws_a, ws_b, csem):
    core = lax.axis_index("c")
    sub = lax.axis_index("s")
    wid = sub * NC + core

    pltpu.sync_copy(rows_h, rows_v)
    pltpu.sync_copy(s_h, sv)

    iota = lax.broadcasted_iota(jnp.int32, (16,), 0)

    def unit_params(u):
        slot = wid * R_PER + u // NSLAB
        b = _slot_val(rows_v, slot)
        s = _slot_val(sv, slot)
        rc = u % NSLAB
        return b, s, rc

    def rd(u, buf, sem):
        b, s, rc = unit_params(u)

        @pl.when(s > 0)
        def _():
            pltpu.async_copy(x_h.at[b, rc], buf, sem)

        @pl.when(s == 0)
        def _():
            pltpu.async_copy(x_h.at[b, rc], out_h.at[b, rc], csem)

    def wait_rd(buf, sem):
        pltpu.make_async_copy(x_h.at[0, 0], buf, sem).wait()

    def wait_wr(buf, sem):
        pltpu.make_async_copy(buf, out_h.at[0, 0], sem).wait()

    def wait_rd_if(u, buf, sem):
        _, s, _ = unit_params(u)

        @pl.when(s > 0)
        def _():
            wait_rd(buf, sem)

    def wait_wr_if(u, buf, sem):
        _, s, _ = unit_params(u)

        @pl.when(s > 0)
        def _():
            wait_wr(buf, sem)

    def process(u, buf, wsem):
        b, s, rc = unit_params(u)

        @pl.when(s > 0)
        def _():
            q = s // 128
            r = s - q * 128

            splats = []
            for ci in range(8):
                def jbody(k, a, ci=ci):
                    for tv in range(8):
                        a = a + buf[k, ci, pl.ds(tv * 16, 16)]
                    return a
                acc = lax.fori_loop(0, NT, jbody,
                                    jnp.zeros((16,), jnp.float32))
                splats.append(jnp.full((16,), _hsum(acc) * INV_T,
                                       jnp.float32))

            h = r // 16
            g = r - h * 16
            idx_hi = jnp.maximum(iota - g, 0)
            idx_lo = jnp.minimum(iota + 16 - g, 15)
            lane_ge_g = iota >= g

            def kbody(j, _):
                kk = (NT - 1) - j
                mh_base = (kk - q) * 8 - h

                def blend(m_hi, m_lo, ci):
                    hi = buf[m_hi // 8, ci, pl.ds((m_hi % 8) * 16, 16)]
                    lo = buf[m_lo // 8, ci, pl.ds((m_lo % 8) * 16, 16)]
                    return jnp.where(lane_ge_g, _shuffle(hi, idx_hi),
                                     _shuffle(lo, idx_lo))

                @pl.when(kk > q)
                def _():
                    for ci in range(8):
                        vs = [blend(mh_base + tv, mh_base + tv - 1, ci)
                              for tv in range(8)]
                        for tv in range(8):
                            buf[kk, ci, pl.ds(tv * 16, 16)] = vs[tv]

                @pl.when(kk <= q)
                def _():
                    for ci in range(8):
                        vs = []
                        for tv in range(8):
                            m_hi = jnp.maximum(mh_base + tv, 0)
                            m_lo = jnp.maximum(mh_base + tv - 1, 0)
                            v = blend(m_hi, m_lo, ci)
                            keep = (128 * kk + tv * 16 + iota) >= s
                            vs.append(jnp.where(keep, v, splats[ci]))
                        for tv in range(8):
                            buf[kk, ci, pl.ds(tv * 16, 16)] = vs[tv]
                return 0

            lax.fori_loop(0, NT, kbody, 0)
            pltpu.async_copy(buf, out_h.at[b, rc], wsem)

    rd(0, buf_a, rs_a)
    rd(1, buf_b, rs_b)

    def pbody(p, _):
        u0 = 2 * p
        wait_rd_if(u0, buf_a, rs_a)
        process(u0, buf_a, ws_a)
        wait_wr_if(u0, buf_a, ws_a)

        @pl.when(u0 + 2 < UNITS)
        def _():
            rd(u0 + 2, buf_a, rs_a)
        wait_rd_if(u0 + 1, buf_b, rs_b)
        process(u0 + 1, buf_b, ws_b)
        wait_wr_if(u0 + 1, buf_b, ws_b)

        @pl.when(u0 + 3 < UNITS)
        def _():
            rd(u0 + 3, buf_b, rs_b)
        return 0

    lax.fori_loop(0, UNITS // 2, pbody, 0)

    def dbody(u, _):
        _, s, _ = unit_params(u)

        @pl.when(s == 0)
        def _():
            pltpu.make_async_copy(x_h.at[0, 0], out_h.at[0, 0], csem).wait()
        return 0

    lax.fori_loop(0, UNITS, dbody, 0)


@jax.jit
def _sc_call(x5, rows_slot, s_slot):
    kfn = pl.kernel(
        _sc_body,
        out_type=jax.ShapeDtypeStruct((B, NSLAB, NT, 8, 128), jnp.float32),
        mesh=plsc.VectorSubcoreMesh(
            core_axis_name="c", subcore_axis_name="s",
            num_cores=NC, num_subcores=NS),
        scratch_types=[
            pltpu.VMEM((B + SLOT_PAD,), jnp.int32),
            pltpu.VMEM((B + SLOT_PAD,), jnp.int32),
            pltpu.VMEM((NT, 8, 128), jnp.float32),
            pltpu.VMEM((NT, 8, 128), jnp.float32),
            pltpu.SemaphoreType.DMA,
            pltpu.SemaphoreType.DMA,
            pltpu.SemaphoreType.DMA,
            pltpu.SemaphoreType.DMA,
            pltpu.SemaphoreType.DMA,
        ],
    )
    return kfn(x5, rows_slot, s_slot)


_SLOT_POS = np.array([(s % R_PER) * NW + s // R_PER for s in range(B)],
                     dtype=np.int32)


def _aug_metadata():
    with jax.default_device(jax.local_devices(backend="cpu")[0]):
        k1, k2 = jax.random.split(jax.random.key(42))
        mask = np.asarray(jax.random.uniform(k1, (B,))) < LIKELIHOOD
        max_steps = int(MAX_SHIFT * float(T))
        shifts = np.asarray(jax.random.randint(k2, (B,), 0, max_steps + 1,
                                               dtype=jnp.int32))
    shifts = np.where(mask, shifts, 0).astype(np.int32)
    order = np.argsort((shifts == 0).astype(np.int32), kind="stable")
    rows_slot = order[_SLOT_POS].astype(np.int32)
    s_slot = shifts[rows_slot].astype(np.int32)
    pad = np.zeros((SLOT_PAD,), np.int32)
    return (np.concatenate([rows_slot, pad]),
            np.concatenate([s_slot, pad]))


_ROWS_SLOT, _S_SLOT = _aug_metadata()


def kernel(x):
    assert x.shape == (B, T, C)
    x5 = (x.transpose(0, 2, 1)
          .reshape(B, NSLAB, 8, NT, 128)
          .transpose(0, 1, 3, 2, 4))
    out5 = _sc_call(x5, jnp.asarray(_ROWS_SLOT), jnp.asarray(_S_SLOT))
    return (out5.transpose(0, 1, 3, 2, 4)
            .reshape(B, C, T)
            .transpose(0, 2, 1))


# speedup vs baseline: 25.7749x; 25.7749x over previous
import jax
import jax.numpy as jnp
import numpy as np
from jax import lax
from jax.experimental import pallas as pl
from jax.experimental.pallas import tpu as pltpu
from jax.experimental.pallas import tpu_sc as plsc

B, T, C = 256, 4096, 64
MAX_SHIFT = 0.1
LIKELIHOOD = 0.2

NC, NS = 2, 16
NW = NC * NS
R_PER = B // NW
NSLAB = C // 8
UNITS = R_PER * NSLAB
NT = T // 128
INV_T = 1.0 / float(T)
SLOT_PAD = 32


def _slot_val(ref, slot):
    return ref[pl.ds(slot, 16)][0]


_GDN = lax.GatherDimensionNumbers(
    offset_dims=(), collapsed_slice_dims=(0,), start_index_map=(0,))


def _shuffle(v, idx):
    return lax.gather(v, idx[:, None], _GDN, (1,),
                      mode=lax.GatherScatterMode.PROMISE_IN_BOUNDS)


def _hsum(v):
    iota = lax.broadcasted_iota(jnp.int32, (16,), 0)
    s = v + lax.rev(v, (0,))
    for m in (4, 2, 1):
        s = s + _shuffle(s, iota ^ m)
    return s[0]


def _sc_body(x_h, rows_h, s_h, out_h, rows_v, sv, buf_a, buf_b,
             rs_a, rs_b, ws_a, ws_b):
    core = lax.axis_index("c")
    sub = lax.axis_index("s")
    wid = sub * NC + core

    pltpu.sync_copy(rows_h, rows_v)
    pltpu.sync_copy(s_h, sv)

    iota = lax.broadcasted_iota(jnp.int32, (16,), 0)

    def unit_params(u):
        slot = wid * R_PER + u // NSLAB
        b = _slot_val(rows_v, slot)
        s = _slot_val(sv, slot)
        rc = u % NSLAB
        return b, s, rc

    def rd(u, buf, sem):
        b, _, rc = unit_params(u)
        pltpu.async_copy(x_h.at[b, rc], buf, sem)

    def wait_rd(buf, sem):
        pltpu.make_async_copy(x_h.at[0, 0], buf, sem).wait()

    def wait_wr(buf, sem):
        pltpu.make_async_copy(buf, out_h.at[0, 0], sem).wait()

    def process(u, buf, wsem):
        b, s, rc = unit_params(u)

        @pl.when(s > 0)
        def _():
            q = s // 128
            r = s - q * 128

            splats = []
            for ci in range(8):
                def jbody(k, a, ci=ci):
                    for tv in range(8):
                        a = a + buf[k, ci, pl.ds(tv * 16, 16)]
                    return a
                acc = lax.fori_loop(0, NT, jbody,
                                    jnp.zeros((16,), jnp.float32))
                splats.append(jnp.full((16,), _hsum(acc) * INV_T,
                                       jnp.float32))

            h = r // 16
            g = r - h * 16
            idx_hi = jnp.maximum(iota - g, 0)
            idx_lo = jnp.minimum(iota + 16 - g, 15)
            lane_ge_g = iota >= g

            def kbody(j, _):
                kk = (NT - 1) - j
                mh_base = (kk - q) * 8 - h

                def blend(m_hi, m_lo, ci):
                    hi = buf[m_hi // 8, ci, pl.ds((m_hi % 8) * 16, 16)]
                    lo = buf[m_lo // 8, ci, pl.ds((m_lo % 8) * 16, 16)]
                    return jnp.where(lane_ge_g, _shuffle(hi, idx_hi),
                                     _shuffle(lo, idx_lo))

                @pl.when(kk > q)
                def _():
                    for ci in range(8):
                        vs = [blend(mh_base + tv, mh_base + tv - 1, ci)
                              for tv in range(8)]
                        for tv in range(8):
                            buf[kk, ci, pl.ds(tv * 16, 16)] = vs[tv]

                @pl.when(kk <= q)
                def _():
                    for ci in range(8):
                        vs = []
                        for tv in range(8):
                            m_hi = jnp.maximum(mh_base + tv, 0)
                            m_lo = jnp.maximum(mh_base + tv - 1, 0)
                            v = blend(m_hi, m_lo, ci)
                            keep = (128 * kk + tv * 16 + iota) >= s
                            vs.append(jnp.where(keep, v, splats[ci]))
                        for tv in range(8):
                            buf[kk, ci, pl.ds(tv * 16, 16)] = vs[tv]
                return 0

            lax.fori_loop(0, NT, kbody, 0)

        pltpu.async_copy(buf, out_h.at[b, rc], wsem)

    rd(0, buf_a, rs_a)
    rd(1, buf_b, rs_b)

    def pbody(p, _):
        u0 = 2 * p
        wait_rd(buf_a, rs_a)
        process(u0, buf_a, ws_a)
        wait_wr(buf_a, ws_a)

        @pl.when(u0 + 2 < UNITS)
        def _():
            rd(u0 + 2, buf_a, rs_a)
        wait_rd(buf_b, rs_b)
        process(u0 + 1, buf_b, ws_b)
        wait_wr(buf_b, ws_b)

        @pl.when(u0 + 3 < UNITS)
        def _():
            rd(u0 + 3, buf_b, rs_b)
        return 0

    lax.fori_loop(0, UNITS // 2, pbody, 0)


@jax.jit
def _sc_call(x5, rows_slot, s_slot):
    kfn = pl.kernel(
        _sc_body,
        out_type=jax.ShapeDtypeStruct((B, NSLAB, NT, 8, 128), jnp.float32),
        mesh=plsc.VectorSubcoreMesh(
            core_axis_name="c", subcore_axis_name="s",
            num_cores=NC, num_subcores=NS),
        scratch_types=[
            pltpu.VMEM((B + SLOT_PAD,), jnp.int32),
            pltpu.VMEM((B + SLOT_PAD,), jnp.int32),
            pltpu.VMEM((NT, 8, 128), jnp.float32),
            pltpu.VMEM((NT, 8, 128), jnp.float32),
            pltpu.SemaphoreType.DMA,
            pltpu.SemaphoreType.DMA,
            pltpu.SemaphoreType.DMA,
            pltpu.SemaphoreType.DMA,
        ],
    )
    return kfn(x5, rows_slot, s_slot)


_SLOT_POS = np.array([(s % R_PER) * NW + s // R_PER for s in range(B)],
                     dtype=np.int32)


def _aug_metadata():
    with jax.default_device(jax.local_devices(backend="cpu")[0]):
        k1, k2 = jax.random.split(jax.random.key(42))
        mask = np.asarray(jax.random.uniform(k1, (B,))) < LIKELIHOOD
        max_steps = int(MAX_SHIFT * float(T))
        shifts = np.asarray(jax.random.randint(k2, (B,), 0, max_steps + 1,
                                               dtype=jnp.int32))
    shifts = np.where(mask, shifts, 0).astype(np.int32)
    order = np.argsort((shifts == 0).astype(np.int32), kind="stable")
    rows_slot = order[_SLOT_POS].astype(np.int32)
    s_slot = shifts[rows_slot].astype(np.int32)
    pad = np.zeros((SLOT_PAD,), np.int32)
    return (np.concatenate([rows_slot, pad]),
            np.concatenate([s_slot, pad]))


_ROWS_SLOT, _S_SLOT = _aug_metadata()


def kernel(x):
    assert x.shape == (B, T, C)
    x5 = (x.transpose(0, 2, 1)
          .reshape(B, NSLAB, 8, NT, 128)
          .transpose(0, 1, 3, 2, 4))
    out5 = _sc_call(x5, jnp.asarray(_ROWS_SLOT), jnp.asarray(_S_SLOT))
    return (out5.transpose(0, 1, 3, 2, 4)
            .reshape(B, C, T)
            .transpose(0, 2, 1))
